# Initial kernel scaffold; baseline (speedup 1.0000x reference)
import jax
import jax.numpy as jnp
from jax.experimental import pallas as pl


def kernel(inp, hx, w_qs, w_ks, w_vs, W_ih, W_hh, b_ih, b_hh, step):
    raise NotImplementedError("write your pallas kernel here")


# speedup vs baseline: 1.4282x; 1.4282x over previous
import jax
import jax.numpy as jnp
import numpy as np
from jax.experimental import pallas as pl
from jax.experimental.pallas import tpu as pltpu

B = 1024
NINP = 512
NHID = 1024
NBLK = 16
TOPK = 8
BLK = NHID // NBLK
ATT = 4 * BLK
GIN = ATT * NBLK
DK = 64
JT = 128
NJ = NHID // JT


def _body(inp_ref, hx_ref, p0_ref, p1_ref, wvs_ref,
          wir_ref, wiz_ref, win_ref, whr_ref, whz_ref, whn_ref,
          bih_ref, bhh_ref,
          hx_out_ref, mask_out_ref,
          att_sc, m16_sc):
    j = pl.program_id(0)

    @pl.when(j == 0)
    def _prep():
        vv1 = jnp.dot(inp_ref[...], wvs_ref[...],
                      preferred_element_type=jnp.float32)
        p1 = p1_ref[...]
        for blk in range(NBLK):
            att_sc[:, blk * ATT:(blk + 1) * ATT] = (
                p1[:, blk:blk + 1] * vv1).astype(jnp.bfloat16)
        p0 = p0_ref[...]
        colid = jax.lax.broadcasted_iota(jnp.int32, (B, NBLK), 1)
        mcols = []
        for i in range(NBLK):
            vi = p0[:, i:i + 1]
            gt = jnp.sum(jnp.where(p0 > vi, 1.0, 0.0), axis=1, keepdims=True)
            eqb = jnp.sum(jnp.where((p0 == vi) & (colid < i), 1.0, 0.0),
                          axis=1, keepdims=True)
            mcols.append(jnp.where(gt + eqb >= float(NBLK - TOPK), 1.0, 0.0))
        m16_sc[...] = jnp.concatenate(mcols, axis=1)

    dn = (((1,), (1,)), ((), ()))
    attb = att_sc[...]
    hxb = hx_ref[...].astype(jnp.bfloat16)
    c0 = j * JT

    def gate(wi_ref, wh_ref, g):
        gi = jax.lax.dot_general(attb, wi_ref[...].astype(jnp.bfloat16), dn,
                                 preferred_element_type=jnp.float32)
        gh = jax.lax.dot_general(hxb, wh_ref[...].astype(jnp.bfloat16), dn,
                                 preferred_element_type=jnp.float32)
        bi = bih_ref[0:1, pl.ds(g * NHID + c0, JT)]
        bh = bhh_ref[0:1, pl.ds(g * NHID + c0, JT)]
        return gi + bi, gh + bh

    i_r, h_r = gate(wir_ref, whr_ref, 0)
    i_z, h_z = gate(wiz_ref, whz_ref, 1)
    i_n, h_n = gate(win_ref, whn_ref, 2)
    r = jax.nn.sigmoid(i_r + h_r)
    z = jax.nn.sigmoid(i_z + h_z)
    n = jnp.tanh(i_n + r * h_n)
    hxt = hx_ref[:, pl.ds(c0, JT)]
    hx_new = (1.0 - z) * n + z * hxt
    erow = jax.lax.broadcasted_iota(jnp.int32, (NBLK, JT), 0)
    ecol = jax.lax.broadcasted_iota(jnp.int32, (NBLK, JT), 1) // BLK
    expand = jnp.where(erow == ecol + j * (JT // BLK), 1.0, 0.0)
    mt = jnp.dot(m16_sc[...], expand, preferred_element_type=jnp.float32)
    hx_out_ref[...] = mt * hx_new + (1.0 - mt) * hxt
    mask_out_ref[...] = mt


def kernel(inp, hx, w_qs, w_ks, w_vs, W_ih, W_hh, b_ih, b_hh, step):
    del step
    b = inp.shape[0]
    inp_use = inp.reshape(b, 1, NINP)
    inp_use = jnp.concatenate([jnp.zeros_like(inp_use), inp_use], axis=1)
    q = jnp.einsum('bnd,nde->bne', hx.reshape(b, NBLK, BLK), w_qs)
    kk = jnp.einsum('bnd,nde->bne', inp_use, w_ks)
    iatt = jax.nn.softmax(jnp.einsum('bqd,bkd->bqk', q, kk) / np.sqrt(DK),
                          axis=-1)
    p0 = iatt[:, :, 0]
    p1 = iatt[:, :, 1]

    wvs1 = w_vs[1]
    bih2 = b_ih.reshape(1, 3 * NHID)
    bhh2 = b_hh.reshape(1, 3 * NHID)

    in_specs = [
        pl.BlockSpec((B, NINP), lambda j: (0, 0)),
        pl.BlockSpec((B, NHID), lambda j: (0, 0)),
        pl.BlockSpec((B, NBLK), lambda j: (0, 0)),
        pl.BlockSpec((B, NBLK), lambda j: (0, 0)),
        pl.BlockSpec((NINP, ATT), lambda j: (0, 0)),
        pl.BlockSpec((JT, GIN), lambda j: (j, 0)),
        pl.BlockSpec((JT, GIN), lambda j: (NJ + j, 0)),
        pl.BlockSpec((JT, GIN), lambda j: (2 * NJ + j, 0)),
        pl.BlockSpec((JT, NHID), lambda j: (j, 0)),
        pl.BlockSpec((JT, NHID), lambda j: (NJ + j, 0)),
        pl.BlockSpec((JT, NHID), lambda j: (2 * NJ + j, 0)),
        pl.BlockSpec((1, 3 * NHID), lambda j: (0, 0)),
        pl.BlockSpec((1, 3 * NHID), lambda j: (0, 0)),
    ]
    out_specs = [
        pl.BlockSpec((B, JT), lambda j: (0, j)),
        pl.BlockSpec((B, JT), lambda j: (0, j)),
    ]
    hx_out, mask = pl.pallas_call(
        _body,
        grid=(NJ,),
        in_specs=in_specs,
        out_specs=out_specs,
        out_shape=[
            jax.ShapeDtypeStruct((B, NHID), jnp.float32),
            jax.ShapeDtypeStruct((B, NHID), jnp.float32),
        ],
        scratch_shapes=[
            pltpu.VMEM((B, GIN), jnp.bfloat16),
            pltpu.VMEM((B, NBLK), jnp.float32),
        ],
        compiler_params=pltpu.CompilerParams(
            dimension_semantics=("arbitrary",),
            vmem_limit_bytes=100 * 1024 * 1024,
        ),
    )(inp, hx, p0, p1, wvs1, W_ih, W_ih, W_ih, W_hh, W_hh, W_hh,
      bih2, bhh2)
    return hx_out, mask


# speedup vs baseline: 1.8743x; 1.3123x over previous
import jax
import jax.numpy as jnp
import numpy as np
from jax.experimental import pallas as pl
from jax.experimental.pallas import tpu as pltpu

B = 1024
NINP = 512
NHID = 1024
NBLK = 16
TOPK = 8
BLK = NHID // NBLK
ATT = 4 * BLK
GIN = ATT * NBLK
DK = 64
GT = 256
NG = 3 * NHID // GT
NOUT = NHID // GT


def _body(inp_ref, hx_ref, p0_ref, p1_ref, wvs_ref, wih_ref, whh_ref,
          bih_ref, bhh_ref,
          hx_out_ref, mask_out_ref,
          att_sc, hxb_sc, gates_sc, m16_sc):
    s = pl.program_id(0)

    @pl.when(s == 0)
    def _prep():
        hxb_sc[...] = hx_ref[...].astype(jnp.bfloat16)
        vv1 = jnp.dot(inp_ref[...], wvs_ref[...],
                      preferred_element_type=jnp.float32)
        p1 = p1_ref[...]
        for blk in range(NBLK):
            att_sc[:, blk * ATT:(blk + 1) * ATT] = (
                p1[:, blk:blk + 1] * vv1).astype(jnp.bfloat16)
        p0 = p0_ref[...]
        colid = jax.lax.broadcasted_iota(jnp.int32, (B, NBLK), 1)
        mcols = []
        for i in range(NBLK):
            vi = p0[:, i:i + 1]
            gt = jnp.sum(jnp.where(p0 > vi, 1.0, 0.0), axis=1, keepdims=True)
            eqb = jnp.sum(jnp.where((p0 == vi) & (colid < i), 1.0, 0.0),
                          axis=1, keepdims=True)
            mcols.append(jnp.where(gt + eqb >= float(NBLK - TOPK), 1.0, 0.0))
        m16_sc[...] = jnp.concatenate(mcols, axis=1)

    dn = (((1,), (1,)), ((), ()))
    gih = jax.lax.dot_general(att_sc[...],
                              wih_ref[...].astype(jnp.bfloat16), dn,
                              preferred_element_type=jnp.float32)
    ghh = jax.lax.dot_general(hxb_sc[...],
                              whh_ref[...].astype(jnp.bfloat16), dn,
                              preferred_element_type=jnp.float32)

    @pl.when(s < NG - NOUT)
    def _stash():
        gates_sc[:, pl.ds(s * GT, GT)] = (gih + ghh).astype(jnp.bfloat16)

    @pl.when(s >= NG - NOUT)
    def _finish():
        t = s - (NG - NOUT)
        c0 = t * GT

        def pre(g):
            gsum = gates_sc[:, pl.ds(g * NHID + c0, GT)].astype(jnp.float32)
            bi = bih_ref[0:1, pl.ds(g * NHID + c0, GT)]
            bh = bhh_ref[0:1, pl.ds(g * NHID + c0, GT)]
            return gsum + (bi + bh)

        r = jax.nn.sigmoid(pre(0))
        z = jax.nn.sigmoid(pre(1))
        gi_n = gih + bih_ref[0:1, pl.ds(2 * NHID + c0, GT)]
        gh_n = ghh + bhh_ref[0:1, pl.ds(2 * NHID + c0, GT)]
        n = jnp.tanh(gi_n + r * gh_n)
        hxt = hx_ref[:, pl.ds(c0, GT)]
        hx_new = (1.0 - z) * n + z * hxt
        erow = jax.lax.broadcasted_iota(jnp.int32, (NBLK, GT), 0)
        ecol = jax.lax.broadcasted_iota(jnp.int32, (NBLK, GT), 1) // BLK
        expand = jnp.where(erow == ecol + t * (GT // BLK), 1.0, 0.0)
        mt = jnp.dot(m16_sc[...], expand, preferred_element_type=jnp.float32)
        hx_out_ref[:, pl.ds(c0, GT)] = mt * hx_new + (1.0 - mt) * hxt
        mask_out_ref[:, pl.ds(c0, GT)] = mt


def kernel(inp, hx, w_qs, w_ks, w_vs, W_ih, W_hh, b_ih, b_hh, step):
    del step
    b = inp.shape[0]
    inp_use = inp.reshape(b, 1, NINP)
    inp_use = jnp.concatenate([jnp.zeros_like(inp_use), inp_use], axis=1)
    q = jnp.einsum('bnd,nde->bne', hx.reshape(b, NBLK, BLK), w_qs)
    kk = jnp.einsum('bnd,nde->bne', inp_use, w_ks)
    iatt = jax.nn.softmax(jnp.einsum('bqd,bkd->bqk', q, kk) / np.sqrt(DK),
                          axis=-1)
    p0 = iatt[:, :, 0]
    p1 = iatt[:, :, 1]

    wvs1 = w_vs[1]
    bih2 = b_ih.reshape(1, 3 * NHID)
    bhh2 = b_hh.reshape(1, 3 * NHID)

    in_specs = [
        pl.BlockSpec((B, NINP), lambda s: (0, 0)),
        pl.BlockSpec((B, NHID), lambda s: (0, 0)),
        pl.BlockSpec((B, NBLK), lambda s: (0, 0)),
        pl.BlockSpec((B, NBLK), lambda s: (0, 0)),
        pl.BlockSpec((NINP, ATT), lambda s: (0, 0)),
        pl.BlockSpec((GT, GIN), lambda s: (s, 0)),
        pl.BlockSpec((GT, NHID), lambda s: (s, 0)),
        pl.BlockSpec((1, 3 * NHID), lambda s: (0, 0)),
        pl.BlockSpec((1, 3 * NHID), lambda s: (0, 0)),
    ]
    out_specs = [
        pl.BlockSpec((B, NHID), lambda s: (0, 0)),
        pl.BlockSpec((B, NHID), lambda s: (0, 0)),
    ]
    hx_out, mask = pl.pallas_call(
        _body,
        grid=(NG,),
        in_specs=in_specs,
        out_specs=out_specs,
        out_shape=[
            jax.ShapeDtypeStruct((B, NHID), jnp.float32),
            jax.ShapeDtypeStruct((B, NHID), jnp.float32),
        ],
        scratch_shapes=[
            pltpu.VMEM((B, GIN), jnp.bfloat16),
            pltpu.VMEM((B, NHID), jnp.bfloat16),
            pltpu.VMEM((B, 2 * NHID), jnp.bfloat16),
            pltpu.VMEM((B, NBLK), jnp.float32),
        ],
        compiler_params=pltpu.CompilerParams(
            dimension_semantics=("arbitrary",),
            vmem_limit_bytes=63 * 1024 * 1024,
        ),
    )(inp, hx, p0, p1, wvs1, W_ih, W_hh, bih2, bhh2)
    return hx_out, mask
